# baseline (speedup 1.0000x reference)
import functools

import jax
import jax.numpy as jnp
from jax.experimental import pallas as pl
from jax.experimental.pallas import tpu as pltpu

TILE_N = 2048


def _embed_kernel(day_ref, week_ref, day_tab_ref, week_tab_ref, out_ref,
                  *scratch, time_steps, n_weeks, t_row, bb, manual_out):
    tile_n = day_ref.shape[-1]

    table = jnp.concatenate(
        [day_tab_ref[...], week_tab_ref[...]], axis=0).astype(jnp.bfloat16)

    iota_d = jax.lax.broadcasted_iota(jnp.int32, (time_steps, tile_n), 0)
    iota_w = jax.lax.broadcasted_iota(jnp.int32, (8, tile_n), 0)

    results = []
    for i in range(bb):
        day = day_ref[i, t_row:t_row + 1, :]
        week = week_ref[i, t_row:t_row + 1, :]

        day_idx = jnp.clip((day * float(time_steps)).astype(jnp.int32),
                           0, time_steps - 1)
        week_idx = jnp.clip(week.astype(jnp.int32), 0, n_weeks - 1)

        onehot = jnp.concatenate(
            [(iota_d == day_idx).astype(jnp.bfloat16),
             (iota_w == week_idx).astype(jnp.bfloat16)], axis=0)

        res = jax.lax.dot_general(
            table, onehot, (((0,), (0,)), ((), ())),
            preferred_element_type=jnp.float32)
        results.append(res)

    if not manual_out:
        for i in range(bb):
            out_ref[i] = results[i][:, None, :]
        return

    buf0, buf1, sem0, sem1 = scratch
    n2 = pl.num_programs(1)
    c = pl.program_id(0)
    j = pl.program_id(1)
    base = (c * n2 + j) * bb
    nq = bb // 2

    def copies(buf, sem, dst_base):
        return [pltpu.make_async_copy(
                    buf.at[pl.ds(2 * q, 2)],
                    out_ref.at[pl.ds(dst_base + 2 * q, 2)],
                    sem.at[q]) for q in range(nq)]

    for slot, buf, sem in ((0, buf0, sem0), (1, buf1, sem1)):
        @pl.when(j % 2 == slot)
        def _(buf=buf, sem=sem):
            @pl.when(j >= 2)
            def _():
                for cp in copies(buf, sem, base - 2 * bb):
                    cp.wait()
            for i in range(bb):
                buf[i] = results[i][:, None, :]
            for cp in copies(buf, sem, base):
                cp.start()

    @pl.when(j == n2 - 1)
    def _():
        for cp in copies(buf0, sem0, base):
            cp.wait()
        for cp in copies(buf1, sem1, base):
            cp.wait()


def kernel(x, time_day, time_week):
    B, T, N, C = x.shape
    time_steps, F = time_day.shape
    n_weeks = time_week.shape[0]

    week_tab = jnp.concatenate(
        [time_week, jnp.zeros((8 - n_weeks, F), time_week.dtype)], axis=0)

    n_pad = ((N + TILE_N - 1) // TILE_N) * TILE_N
    n_tiles = n_pad // TILE_N

    if n_pad == N and B % 8 == 0 and T % 8 == 0:
        bb = 8
        t_blk = (T - 1) // 8
        t_row = (T - 1) % 8
        xt = jnp.transpose(x.astype(jnp.float32), (0, 3, 1, 2))
        day_in = xt
        week_in = xt
        day_spec = pl.BlockSpec((bb, None, 8, TILE_N),
                                lambda b, n: (b, 1, t_blk, n))
        week_spec = pl.BlockSpec((bb, None, 8, TILE_N),
                                 lambda b, n: (b, 2, t_blk, n))
    else:
        bb = 1
        t_row = 0
        day_in = x[:, -1:, :, 1].astype(jnp.float32)
        week_in = x[:, -1:, :, 2].astype(jnp.float32)
        if n_pad != N:
            day_in = jnp.pad(day_in, ((0, 0), (0, 0), (0, n_pad - N)))
            week_in = jnp.pad(week_in, ((0, 0), (0, 0), (0, n_pad - N)))
        day_spec = pl.BlockSpec((bb, 1, TILE_N), lambda b, n: (b, 0, n))
        week_spec = pl.BlockSpec((bb, 1, TILE_N), lambda b, n: (b, 0, n))

    n_groups = B // bb
    manual_out = (bb % 2 == 0 and n_tiles == 1
                  and n_groups % 2 == 0 and n_groups >= 4)
    body = functools.partial(_embed_kernel, time_steps=time_steps,
                             n_weeks=n_weeks, t_row=t_row, bb=bb,
                             manual_out=manual_out)

    if manual_out:
        n2 = n_groups // 2
        grid = (2, n2)
        group_idx = lambda c, j: c * n2 + j
        out_spec = pl.BlockSpec(memory_space=pl.ANY)
        scratch_shapes = [pltpu.VMEM((bb, F, 1, TILE_N), jnp.float32),
                          pltpu.VMEM((bb, F, 1, TILE_N), jnp.float32),
                          pltpu.SemaphoreType.DMA((bb // 2,)),
                          pltpu.SemaphoreType.DMA((bb // 2,))]
        semantics = ("parallel", "arbitrary")
    else:
        grid = (n_groups, n_tiles)
        group_idx = lambda b, n: b
        out_spec = pl.BlockSpec((bb, F, 1, TILE_N), lambda b, n: (b, 0, 0, n))
        scratch_shapes = []
        semantics = ("parallel", "parallel")

    if n_pad == N and B % 8 == 0 and T % 8 == 0:
        day_spec = pl.BlockSpec((bb, None, 8, TILE_N),
                                lambda a, b: (group_idx(a, b), 1, t_blk,
                                              0 if manual_out else b))
        week_spec = pl.BlockSpec((bb, None, 8, TILE_N),
                                 lambda a, b: (group_idx(a, b), 2, t_blk,
                                               0 if manual_out else b))

    out = pl.pallas_call(
        body,
        out_shape=jax.ShapeDtypeStruct((B, F, 1, n_pad), jnp.float32),
        grid=grid,
        in_specs=[
            day_spec,
            week_spec,
            pl.BlockSpec((time_steps, F), lambda a, b: (0, 0)),
            pl.BlockSpec((8, F), lambda a, b: (0, 0)),
        ],
        out_specs=out_spec,
        scratch_shapes=scratch_shapes,
        compiler_params=pltpu.CompilerParams(
            dimension_semantics=semantics),
    )(day_in, week_in, time_day, week_tab)

    return jnp.transpose(out[:, :, :, :N], (0, 1, 3, 2))


# speedup vs baseline: 1.2687x; 1.2687x over previous
import functools

import jax
import jax.numpy as jnp
from jax.experimental import pallas as pl
from jax.experimental.pallas import tpu as pltpu

TILE_N = 2048


def _embed_kernel(day_ref, week_ref, day_tab_ref, week_tab_ref, out_ref, *,
                  time_steps, n_weeks, t_row):
    bb = out_ref.shape[0]
    tile_n = out_ref.shape[-1]

    table = jnp.concatenate(
        [day_tab_ref[...], week_tab_ref[...]], axis=0).astype(jnp.bfloat16)

    iota_d = jax.lax.broadcasted_iota(jnp.int32, (time_steps, tile_n), 0)
    iota_w = jax.lax.broadcasted_iota(jnp.int32, (8, tile_n), 0)

    for i in range(bb):
        day = day_ref[i, t_row:t_row + 1, :]
        week = week_ref[i, t_row:t_row + 1, :]

        day_idx = jnp.clip((day * float(time_steps)).astype(jnp.int32),
                           0, time_steps - 1)
        week_idx = jnp.clip(week.astype(jnp.int32), 0, n_weeks - 1)

        onehot = jnp.concatenate(
            [(iota_d == day_idx).astype(jnp.bfloat16),
             (iota_w == week_idx).astype(jnp.bfloat16)], axis=0)

        res = jax.lax.dot_general(
            table, onehot, (((0,), (0,)), ((), ())),
            preferred_element_type=jnp.float32)
        out_ref[i] = res[:, None, :]


def kernel(x, time_day, time_week):
    B, T, N, C = x.shape
    time_steps, F = time_day.shape
    n_weeks = time_week.shape[0]

    week_tab = jnp.concatenate(
        [time_week, jnp.zeros((8 - n_weeks, F), time_week.dtype)], axis=0)

    n_pad = ((N + TILE_N - 1) // TILE_N) * TILE_N
    n_tiles = n_pad // TILE_N

    if n_pad == N and B % 8 == 0 and T % 8 == 0:
        bb = 8
        t_blk = (T - 1) // 8
        t_row = (T - 1) % 8
        xt = jnp.transpose(x.astype(jnp.float32), (0, 3, 1, 2))
        day_in = xt
        week_in = xt
        day_spec = pl.BlockSpec((bb, None, 8, TILE_N),
                                lambda b, n: (b, 1, t_blk, n))
        week_spec = pl.BlockSpec((bb, None, 8, TILE_N),
                                 lambda b, n: (b, 2, t_blk, n))
    else:
        bb = 1
        t_row = 0
        day_in = x[:, -1:, :, 1].astype(jnp.float32)
        week_in = x[:, -1:, :, 2].astype(jnp.float32)
        if n_pad != N:
            day_in = jnp.pad(day_in, ((0, 0), (0, 0), (0, n_pad - N)))
            week_in = jnp.pad(week_in, ((0, 0), (0, 0), (0, n_pad - N)))
        day_spec = pl.BlockSpec((bb, 1, TILE_N), lambda b, n: (b, 0, n))
        week_spec = pl.BlockSpec((bb, 1, TILE_N), lambda b, n: (b, 0, n))

    body = functools.partial(_embed_kernel, time_steps=time_steps,
                             n_weeks=n_weeks, t_row=t_row)

    out = pl.pallas_call(
        body,
        out_shape=jax.ShapeDtypeStruct((B, F, 1, n_pad), jnp.float32),
        grid=(B // bb, n_tiles),
        in_specs=[
            day_spec,
            week_spec,
            pl.BlockSpec((time_steps, F), lambda b, n: (0, 0)),
            pl.BlockSpec((8, F), lambda b, n: (0, 0)),
        ],
        out_specs=pl.BlockSpec((bb, F, 1, TILE_N), lambda b, n: (b, 0, 0, n)),
        compiler_params=pltpu.CompilerParams(
            dimension_semantics=("parallel", "parallel")),
    )(day_in, week_in, time_day, week_tab)

    return jnp.transpose(out[:, :, :, :N], (0, 1, 3, 2))
